# baseline (speedup 1.0000x reference)
import jax
import jax.numpy as jnp
from jax import lax
from jax.experimental import pallas as pl
from jax.experimental.pallas import tpu as pltpu
from jax.experimental.pallas import tpu_sc as plsc

N = 10000
NP = 10240
E = 320000
CH = 128
EPT_F = 20480
EPT_S = 10240
ET_S = 2 * 16 * EPT_S
RPT = NP // 16
DIN = 128
DH = 256


def _make_sc_pass(D, BC, EPT, dtype, spmem_src):
    mesh = plsc.VectorSubcoreMesh(core_axis_name="c", subcore_axis_name="s")
    BR = BC * CH
    T = EPT // BR

    assert T % 2 == 0

    def body(tbl, gidx, sidx, zrows, out, gidx_v, sidx_v, rows_v, acc,
             tbl_sh, gs0, gs1, ss0, ss1):
        cid = lax.axis_index("c")
        sid = lax.axis_index("s")
        wid = cid * 16 + sid
        pltpu.sync_copy(gidx.at[wid], gidx_v)
        pltpu.sync_copy(sidx.at[wid], sidx_v)
        if spmem_src:
            pltpu.sync_copy(tbl.at[pl.ds(cid * NP + sid * RPT, RPT)],
                            tbl_sh.at[pl.ds(sid * RPT, RPT)])
        pltpu.sync_copy(zrows, acc.at[pl.ds(sid * RPT, RPT)])
        plsc.subcore_barrier()
        gsrc = tbl_sh if spmem_src else tbl

        def sg(g, b, sem):
            pltpu.async_copy(gsrc.at[gidx_v.at[g]], rows_v.at[b], sem)

        def wg(b, sem):
            pltpu.make_async_copy(gsrc.at[gidx_v.at[0]], rows_v.at[b],
                                  sem).wait()

        def ss(g, b, sem):
            pltpu.async_copy(rows_v.at[b], acc.at[sidx_v.at[g]], sem,
                             add=True)

        def ws(b, sem):
            pltpu.make_async_copy(rows_v.at[b], acc.at[sidx_v.at[0]],
                                  sem).wait()

        sg(0, 0, gs0)

        def pair(k, carry):
            e = 2 * k
            wg(0, gs0)
            sg(e + 1, 1, gs1)
            ss(e, 0, ss0)
            wg(1, gs1)
            ws(0, ss0)
            sg(e + 2, 0, gs0)
            ss(e + 1, 1, ss1)
            ws(1, ss1)
            return carry

        lax.fori_loop(0, T // 2 - 1, pair, 0)
        wg(0, gs0)
        sg(T - 1, 1, gs1)
        ss(T - 2, 0, ss0)
        wg(1, gs1)
        ws(0, ss0)
        ss(T - 1, 1, ss1)
        ws(1, ss1)
        plsc.subcore_barrier()
        pltpu.sync_copy(acc.at[pl.ds(sid * RPT, RPT)],
                        out.at[pl.ds(cid * NP + sid * RPT, RPT)])

    return pl.kernel(
        body,
        out_type=jax.ShapeDtypeStruct((2 * NP, D), dtype),
        mesh=mesh,
        compiler_params=pltpu.CompilerParams(use_tc_tiling_on_sc=False),
        scratch_types=[
            pltpu.VMEM((T, BR), jnp.int32),
            pltpu.VMEM((T, BR), jnp.int32),
            pltpu.VMEM((2, BR, D), dtype),
            pltpu.VMEM_SHARED((NP, D), dtype),
            pltpu.VMEM_SHARED((NP if spmem_src else 8, D), dtype),
            pltpu.SemaphoreType.DMA,
            pltpu.SemaphoreType.DMA,
            pltpu.SemaphoreType.DMA,
            pltpu.SemaphoreType.DMA,
        ],
    )


import functools


@functools.lru_cache(maxsize=None)
def _get_sc_pass(D, BC, EPT, dtype, spmem_src):
    return _make_sc_pass(D, BC, EPT, dtype, spmem_src)


def _sc_pass128(*args):
    return _get_sc_pass(128, 2, EPT_S, jnp.bfloat16, False)(*args)


def _sc_pass8(*args):
    return _get_sc_pass(8, 4, EPT_S, jnp.float32, False)(*args)


def _norm_tc(deg0, deg1):
    R = 2048

    def body(d0, d1, n_ref, n8_ref):
        i = pl.program_id(0)
        deg = d0[...] + d1[...]
        nrm = lax.rsqrt(jnp.maximum(deg, 1.0))
        n_ref[...] = nrm
        rows = lax.broadcasted_iota(jnp.int32, (R, 1), 0) + i * R
        masked = jnp.where(rows < N, nrm, 0.0)
        n8_ref[...] = masked * jnp.ones((1, 8), jnp.float32)

    return pl.pallas_call(
        body,
        grid=(NP // R,),
        in_specs=[pl.BlockSpec((R, 1), lambda i: (i, 0)),
                  pl.BlockSpec((R, 1), lambda i: (i, 0))],
        out_specs=[pl.BlockSpec((R, 1), lambda i: (i, 0)),
                   pl.BlockSpec((R, 8), lambda i: (i, 0))],
        out_shape=[jax.ShapeDtypeStruct((NP, 1), jnp.float32),
                   jax.ShapeDtypeStruct((NP, 8), jnp.float32)],
    )(deg0, deg1)


def _scale0_tc(h_pad, nrm):
    R = 2048

    def body(h_ref, n_ref, o_ref):
        o_ref[...] = (h_ref[...] * n_ref[...]).astype(jnp.bfloat16)

    return pl.pallas_call(
        body,
        grid=(NP // R,),
        in_specs=[pl.BlockSpec((R, DIN), lambda i: (i, 0)),
                  pl.BlockSpec((R, 1), lambda i: (i, 0))],
        out_specs=pl.BlockSpec((R, DIN), lambda i: (i, 0)),
        out_shape=jax.ShapeDtypeStruct((NP, DIN), jnp.bfloat16),
    )(h_pad, nrm)


def _scale1_tc(t1p0, t1p1, nrm):
    R = 2048

    def body(p0, p1, n_ref, o_ref):
        n = n_ref[...]
        t = p0[...].astype(jnp.float32) + p1[...].astype(jnp.float32)
        o_ref[...] = (t * n * n).astype(jnp.bfloat16)

    return pl.pallas_call(
        body,
        grid=(NP // R,),
        in_specs=[pl.BlockSpec((R, DIN), lambda i: (i, 0)),
                  pl.BlockSpec((R, DIN), lambda i: (i, 0)),
                  pl.BlockSpec((R, 1), lambda i: (i, 0))],
        out_specs=pl.BlockSpec((R, DIN), lambda i: (i, 0)),
        out_shape=jax.ShapeDtypeStruct((NP, DIN), jnp.bfloat16),
    )(t1p0, t1p1, nrm)


def _ctab_tc(g1p0, g1p1, nrm):
    R = 2048

    def body(g0, g1, n_ref, c1_ref, ct_ref):
        n = n_ref[...]
        c1 = n * (g0[...] + g1[...])
        c1_ref[...] = c1
        ct_ref[...] = (n * c1) * jnp.ones((1, 8), jnp.float32)

    return pl.pallas_call(
        body,
        grid=(NP // R,),
        in_specs=[pl.BlockSpec((R, 1), lambda i: (i, 0)),
                  pl.BlockSpec((R, 1), lambda i: (i, 0)),
                  pl.BlockSpec((R, 1), lambda i: (i, 0))],
        out_specs=[pl.BlockSpec((R, 1), lambda i: (i, 0)),
                   pl.BlockSpec((R, 8), lambda i: (i, 0))],
        out_shape=[jax.ShapeDtypeStruct((NP, 1), jnp.float32),
                   jax.ShapeDtypeStruct((NP, 8), jnp.float32)],
    )(g1p0, g1p1, nrm)


def _final_tc(h_pad, t1a, t1b, t2a, t2b, nrm, c1, g2p0, g2p1,
              W1, b1, W2, b2, Wc, bc):
    R = 512
    G = NP // R

    def body(h, t1a_r, t1b_r, t2a_r, t2b_r, n_ref, c1_ref, g20, g21,
             w1, bb1, w2, bb2, wc, bbc, out, acc):
        i = pl.program_id(0)

        @pl.when(i == 0)
        def _():
            acc[...] = jnp.zeros_like(acc)

        n = n_ref[...]

        def mm(x, w):
            return jnp.dot(x, w, preferred_element_type=jnp.float32)

        h1 = (t1a_r[...].astype(jnp.float32)
              + t1b_r[...].astype(jnp.float32)) * n
        h2 = (t2a_r[...].astype(jnp.float32)
              + t2b_r[...].astype(jnp.float32)) * n
        z = mm(h[...], w1[0:128, :])
        z = z + mm(h1, w1[128:256, :])
        z = z + mm(h2, w1[256:384, :])
        y = jnp.maximum(z + bb1[...], 0.0)
        rows = lax.broadcasted_iota(jnp.int32, (R, 1), 0) + i * R
        w0 = jnp.where(rows < N, 1.0, 0.0)
        c2 = n * (g20[...] + g21[...])
        acc[0:1, :] += jnp.sum(y * w0, axis=0, keepdims=True)
        acc[1:2, :] += jnp.sum(y * c1_ref[...], axis=0, keepdims=True)
        acc[2:3, :] += jnp.sum(y * c2, axis=0, keepdims=True)

        @pl.when(i == G - 1)
        def _():
            inv = 1.0 / N
            o = mm(acc[0:1, :] * inv, w2[0:256, :])
            o = o + mm(acc[1:2, :] * inv, w2[256:512, :])
            o = o + mm(acc[2:3, :] * inv, w2[512:768, :])
            o = o + bb2[...]
            out[...] = mm(o, wc[...]) + bbc[...]

    blk = lambda r, c: pl.BlockSpec((r, c), lambda i: (i, 0))
    full = lambda r, c: pl.BlockSpec((r, c), lambda i: (0, 0))

    return pl.pallas_call(
        body,
        grid=(G,),
        in_specs=[blk(R, DIN),
                  blk(R, DIN), blk(R, DIN),
                  blk(R, DIN), blk(R, DIN),
                  blk(R, 1), blk(R, 1), blk(R, 1), blk(R, 1),
                  full(384, DH), full(1, DH), full(768, DH), full(1, DH),
                  full(DH, 10), full(1, 10)],
        out_specs=pl.BlockSpec((1, 10), lambda i: (0, 0)),
        out_shape=jax.ShapeDtypeStruct((1, 10), jnp.float32),
        scratch_shapes=[pltpu.VMEM((8, DH), jnp.float32)],
    )(h_pad, t1a, t1b, t2a, t2b, nrm, c1, g2p0, g2p1,
      W1, b1.reshape(1, DH), W2, b2.reshape(1, DH), Wc, bc.reshape(1, 10))


@jax.jit
def kernel(h, edge_index, W1, b1, W2, b2, Wc, bc):
    src = edge_index[0]
    dst = edge_index[1]

    pad_s = ET_S - E
    TS = EPT_S // 512

    def padded(a, v, w=512):
        return jnp.concatenate([a, jnp.full((pad_s,), v, jnp.int32)]) \
                  .reshape(32, EPT_S // w, w)

    gidx_f = padded(src, N, 256)
    sidx_f = padded(dst, 0, 256)

    gidx_d = padded(jnp.zeros((E,), jnp.int32), 8)
    sidx_d = padded(dst, 0)
    gidx_c = padded(dst, N)
    sidx_c = padded(src, 0)

    h_pad = jnp.pad(h, ((0, NP - N), (0, 0)))
    ones8 = jnp.concatenate([jnp.ones((8, 8), jnp.float32),
                             jnp.zeros((8, 8), jnp.float32)])
    z128 = jnp.zeros((RPT, 128), jnp.bfloat16)
    z8 = jnp.zeros((RPT, 8), jnp.float32)

    deg_p = _sc_pass8(ones8, gidx_d, sidx_d, z8)
    nrm, norm8 = _norm_tc(deg_p[0:NP, 0:1], deg_p[NP:2 * NP, 0:1])

    xs0 = _scale0_tc(h_pad, nrm)
    t1 = _sc_pass128(xs0, gidx_f, sidx_f, z128)
    xs1 = _scale1_tc(t1[0:NP], t1[NP:2 * NP], nrm)
    t2 = _sc_pass128(xs1, gidx_f, sidx_f, z128)

    g1_p = _sc_pass8(norm8, gidx_c, sidx_c, z8)
    c1, ctab2 = _ctab_tc(g1_p[0:NP, 0:1], g1_p[NP:2 * NP, 0:1], nrm)
    g2_p = _sc_pass8(ctab2, gidx_c, sidx_c, z8)

    return _final_tc(h_pad, t1[0:NP], t1[NP:2 * NP], t2[0:NP], t2[NP:2 * NP],
                     nrm, c1, g2_p[0:NP, 0:1], g2_p[NP:2 * NP, 0:1],
                     W1, b1, W2, b2, Wc, bc)


# speedup vs baseline: 1.1644x; 1.1644x over previous
import jax
import jax.numpy as jnp
from jax import lax
from jax.experimental import pallas as pl
from jax.experimental.pallas import tpu as pltpu
from jax.experimental.pallas import tpu_sc as plsc

N = 10000
NP = 10240
E = 320000
CH = 128
EPT_F = 20480
EPT_S = 10240
ET_S = 2 * 16 * EPT_S
RPT = NP // 16
DIN = 128
DH = 256


def _make_sc_pass(D, BC, EPT, dtype, spmem_src):
    mesh = plsc.VectorSubcoreMesh(core_axis_name="c", subcore_axis_name="s")
    BR = BC * CH
    T = EPT // BR

    assert T % 2 == 0

    def body(tbl, gidx, sidx, zrows, out, gidx_v, sidx_v, rows_v, acc,
             tbl_sh, gs0, gs1, ss0, ss1):
        cid = lax.axis_index("c")
        sid = lax.axis_index("s")
        wid = cid * 16 + sid
        pltpu.sync_copy(gidx.at[wid], gidx_v)
        pltpu.sync_copy(sidx.at[wid], sidx_v)
        if spmem_src:
            pltpu.sync_copy(tbl.at[pl.ds(cid * NP + sid * RPT, RPT)],
                            tbl_sh.at[pl.ds(sid * RPT, RPT)])
        pltpu.sync_copy(zrows, acc.at[pl.ds(sid * RPT, RPT)])
        plsc.subcore_barrier()
        gsrc = tbl_sh if spmem_src else tbl

        def sg(g, b, sem):
            pltpu.async_copy(gsrc.at[gidx_v.at[g]], rows_v.at[b], sem)

        def wg(b, sem):
            pltpu.make_async_copy(gsrc.at[gidx_v.at[0]], rows_v.at[b],
                                  sem).wait()

        def ss(g, b, sem):
            pltpu.async_copy(rows_v.at[b], acc.at[sidx_v.at[g]], sem,
                             add=True)

        def ws(b, sem):
            pltpu.make_async_copy(rows_v.at[b], acc.at[sidx_v.at[0]],
                                  sem).wait()

        sg(0, 0, gs0)

        def pair(k, carry):
            e = 2 * k
            wg(0, gs0)
            sg(e + 1, 1, gs1)
            ss(e, 0, ss0)
            wg(1, gs1)
            ws(0, ss0)
            sg(e + 2, 0, gs0)
            ss(e + 1, 1, ss1)
            ws(1, ss1)
            return carry

        lax.fori_loop(0, T // 2 - 1, pair, 0)
        wg(0, gs0)
        sg(T - 1, 1, gs1)
        ss(T - 2, 0, ss0)
        wg(1, gs1)
        ws(0, ss0)
        ss(T - 1, 1, ss1)
        ws(1, ss1)
        plsc.subcore_barrier()
        pltpu.sync_copy(acc.at[pl.ds(sid * RPT, RPT)],
                        out.at[pl.ds(cid * NP + sid * RPT, RPT)])

    return pl.kernel(
        body,
        out_type=jax.ShapeDtypeStruct((2 * NP, D), dtype),
        mesh=mesh,
        compiler_params=pltpu.CompilerParams(use_tc_tiling_on_sc=False),
        scratch_types=[
            pltpu.VMEM((T, BR), jnp.int32),
            pltpu.VMEM((T, BR), jnp.int32),
            pltpu.VMEM((2, BR, D), dtype),
            pltpu.VMEM_SHARED((NP, D), dtype),
            pltpu.VMEM_SHARED((NP if spmem_src else 8, D), dtype),
            pltpu.SemaphoreType.DMA,
            pltpu.SemaphoreType.DMA,
            pltpu.SemaphoreType.DMA,
            pltpu.SemaphoreType.DMA,
        ],
    )


import functools


@functools.lru_cache(maxsize=None)
def _get_sc_pass(D, BC, EPT, dtype, spmem_src):
    return _make_sc_pass(D, BC, EPT, dtype, spmem_src)


def _sc_pass64(*args):
    return _get_sc_pass(64, 2, EPT_F, jnp.bfloat16, True)(*args)


def _sc_pass8(*args):
    return _get_sc_pass(8, 4, EPT_S, jnp.float32, False)(*args)


def _norm_tc(deg0, deg1):
    R = 2048

    def body(d0, d1, n_ref, n8_ref):
        i = pl.program_id(0)
        deg = d0[...] + d1[...]
        nrm = lax.rsqrt(jnp.maximum(deg, 1.0))
        n_ref[...] = nrm
        rows = lax.broadcasted_iota(jnp.int32, (R, 1), 0) + i * R
        masked = jnp.where(rows < N, nrm, 0.0)
        n8_ref[...] = masked * jnp.ones((1, 8), jnp.float32)

    return pl.pallas_call(
        body,
        grid=(NP // R,),
        in_specs=[pl.BlockSpec((R, 1), lambda i: (i, 0)),
                  pl.BlockSpec((R, 1), lambda i: (i, 0))],
        out_specs=[pl.BlockSpec((R, 1), lambda i: (i, 0)),
                   pl.BlockSpec((R, 8), lambda i: (i, 0))],
        out_shape=[jax.ShapeDtypeStruct((NP, 1), jnp.float32),
                   jax.ShapeDtypeStruct((NP, 8), jnp.float32)],
    )(deg0, deg1)


def _scale0_tc(h3, nrm):
    R = 2048
    G = NP // R

    def body(h_ref, n_ref, o_ref):
        c = pl.program_id(0)
        hv = h_ref[...]
        half = jnp.where(c == 0, hv[:, 0, :], hv[:, 1, :])
        o_ref[...] = (half * n_ref[...]).astype(jnp.bfloat16)

    return pl.pallas_call(
        body,
        grid=(2, G),
        in_specs=[pl.BlockSpec((R, 2, 64), lambda c, i: (i, 0, 0)),
                  pl.BlockSpec((R, 1), lambda c, i: (i, 0))],
        out_specs=pl.BlockSpec((R, 64), lambda c, i: (c * G + i, 0)),
        out_shape=jax.ShapeDtypeStruct((2 * NP, 64), jnp.bfloat16),
    )(h3, nrm)


def _scale1_tc(t1, nrm):
    R = 2048
    G = NP // R

    def body(t_ref, n_ref, o_ref):
        n = n_ref[...]
        o_ref[...] = (t_ref[...].astype(jnp.float32) * n * n) \
            .astype(jnp.bfloat16)

    return pl.pallas_call(
        body,
        grid=(2, G),
        in_specs=[pl.BlockSpec((R, 64), lambda c, i: (c * G + i, 0)),
                  pl.BlockSpec((R, 1), lambda c, i: (i, 0))],
        out_specs=pl.BlockSpec((R, 64), lambda c, i: (c * G + i, 0)),
        out_shape=jax.ShapeDtypeStruct((2 * NP, 64), jnp.bfloat16),
    )(t1, nrm)


def _ctab_tc(g1p0, g1p1, nrm):
    R = 2048

    def body(g0, g1, n_ref, c1_ref, ct_ref):
        n = n_ref[...]
        c1 = n * (g0[...] + g1[...])
        c1_ref[...] = c1
        ct_ref[...] = (n * c1) * jnp.ones((1, 8), jnp.float32)

    return pl.pallas_call(
        body,
        grid=(NP // R,),
        in_specs=[pl.BlockSpec((R, 1), lambda i: (i, 0)),
                  pl.BlockSpec((R, 1), lambda i: (i, 0)),
                  pl.BlockSpec((R, 1), lambda i: (i, 0))],
        out_specs=[pl.BlockSpec((R, 1), lambda i: (i, 0)),
                   pl.BlockSpec((R, 8), lambda i: (i, 0))],
        out_shape=[jax.ShapeDtypeStruct((NP, 1), jnp.float32),
                   jax.ShapeDtypeStruct((NP, 8), jnp.float32)],
    )(g1p0, g1p1, nrm)


def _final_tc(h_pad, t1a, t1b, t2a, t2b, nrm, c1, g2p0, g2p1,
              W1, b1, W2, b2, Wc, bc):
    R = 512
    G = NP // R

    def body(h, t1a_r, t1b_r, t2a_r, t2b_r, n_ref, c1_ref, g20, g21,
             w1, bb1, w2, bb2, wc, bbc, out, acc):
        i = pl.program_id(0)

        @pl.when(i == 0)
        def _():
            acc[...] = jnp.zeros_like(acc)

        n = n_ref[...]

        def mm(x, w):
            return jnp.dot(x, w, preferred_element_type=jnp.float32)

        z = mm(h[...], w1[0:128, :])
        z = z + mm(t1a_r[...].astype(jnp.float32) * n, w1[128:192, :])
        z = z + mm(t1b_r[...].astype(jnp.float32) * n, w1[192:256, :])
        z = z + mm(t2a_r[...].astype(jnp.float32) * n, w1[256:320, :])
        z = z + mm(t2b_r[...].astype(jnp.float32) * n, w1[320:384, :])
        y = jnp.maximum(z + bb1[...], 0.0)
        rows = lax.broadcasted_iota(jnp.int32, (R, 1), 0) + i * R
        w0 = jnp.where(rows < N, 1.0, 0.0)
        c2 = n * (g20[...] + g21[...])
        acc[0:1, :] += jnp.sum(y * w0, axis=0, keepdims=True)
        acc[1:2, :] += jnp.sum(y * c1_ref[...], axis=0, keepdims=True)
        acc[2:3, :] += jnp.sum(y * c2, axis=0, keepdims=True)

        @pl.when(i == G - 1)
        def _():
            inv = 1.0 / N
            o = mm(acc[0:1, :] * inv, w2[0:256, :])
            o = o + mm(acc[1:2, :] * inv, w2[256:512, :])
            o = o + mm(acc[2:3, :] * inv, w2[512:768, :])
            o = o + bb2[...]
            out[...] = mm(o, wc[...]) + bbc[...]

    blk = lambda r, c: pl.BlockSpec((r, c), lambda i: (i, 0))
    full = lambda r, c: pl.BlockSpec((r, c), lambda i: (0, 0))

    return pl.pallas_call(
        body,
        grid=(G,),
        in_specs=[blk(R, DIN),
                  blk(R, 64), blk(R, 64),
                  blk(R, 64), blk(R, 64),
                  blk(R, 1), blk(R, 1), blk(R, 1), blk(R, 1),
                  full(384, DH), full(1, DH), full(768, DH), full(1, DH),
                  full(DH, 10), full(1, 10)],
        out_specs=pl.BlockSpec((1, 10), lambda i: (0, 0)),
        out_shape=jax.ShapeDtypeStruct((1, 10), jnp.float32),
        scratch_shapes=[pltpu.VMEM((8, DH), jnp.float32)],
    )(h_pad, t1a, t1b, t2a, t2b, nrm, c1, g2p0, g2p1,
      W1, b1.reshape(1, DH), W2, b2.reshape(1, DH), Wc, bc.reshape(1, 10))


@jax.jit
def kernel(h, edge_index, W1, b1, W2, b2, Wc, bc):
    src = edge_index[0]
    dst = edge_index[1]

    TF = EPT_F // 256
    s16 = src.reshape(16, E // 16)
    d16 = dst.reshape(16, E // 16)
    padf_g = jnp.full((16, EPT_F - E // 16), N, jnp.int32)
    padf_s = jnp.zeros((16, EPT_F - E // 16), jnp.int32)
    g0 = jnp.concatenate([s16, padf_g], axis=1).reshape(16, TF, 256)
    gidx_f = jnp.concatenate([g0, g0], axis=0)
    s0 = jnp.concatenate([d16, padf_s], axis=1).reshape(16, TF, 256)
    sidx_f = jnp.concatenate([s0, s0], axis=0)

    pad_s = ET_S - E
    TS = EPT_S // 512

    def padded(a, v):
        return jnp.concatenate([a, jnp.full((pad_s,), v, jnp.int32)]) \
                  .reshape(32, TS, 512)

    gidx_d = padded(jnp.zeros((E,), jnp.int32), 8)
    sidx_d = padded(dst, 0)
    gidx_c = padded(dst, N)
    sidx_c = padded(src, 0)

    h_pad = jnp.pad(h, ((0, NP - N), (0, 0)))
    h3 = h_pad.reshape(NP, 2, 64)
    ones8 = jnp.concatenate([jnp.ones((8, 8), jnp.float32),
                             jnp.zeros((8, 8), jnp.float32)])
    z64 = jnp.zeros((RPT, 64), jnp.bfloat16)
    z8 = jnp.zeros((RPT, 8), jnp.float32)

    deg_p = _sc_pass8(ones8, gidx_d, sidx_d, z8)
    nrm, norm8 = _norm_tc(deg_p[0:NP, 0:1], deg_p[NP:2 * NP, 0:1])

    xs0 = _scale0_tc(h3, nrm)
    t1 = _sc_pass64(xs0, gidx_f, sidx_f, z64)
    xs1 = _scale1_tc(t1, nrm)
    t2 = _sc_pass64(xs1, gidx_f, sidx_f, z64)

    g1_p = _sc_pass8(norm8, gidx_c, sidx_c, z8)
    c1, ctab2 = _ctab_tc(g1_p[0:NP, 0:1], g1_p[NP:2 * NP, 0:1], nrm)
    g2_p = _sc_pass8(ctab2, gidx_c, sidx_c, z8)

    return _final_tc(h_pad, t1[0:NP], t1[NP:2 * NP], t2[0:NP], t2[NP:2 * NP],
                     nrm, c1, g2_p[0:NP, 0:1], g2_p[NP:2 * NP, 0:1],
                     W1, b1, W2, b2, Wc, bc)
